# Initial kernel scaffold; baseline (speedup 1.0000x reference)
import jax
import jax.numpy as jnp
from jax.experimental import pallas as pl


def kernel(p_t, p0, mask, abar_t, eps_pred, exist_logit):
    raise NotImplementedError("write your pallas kernel here")


# speedup vs baseline: 12.1499x; 12.1499x over previous
import jax
import jax.numpy as jnp
from jax.experimental import pallas as pl
from jax.experimental.pallas import tpu as pltpu

_B, _Q, _T = 64, 1024, 1024


def _loss_kernel(ptx_ref, pty_ref, epsx_ref, epsy_ref, logit_ref,
                 p0x_ref, p0y_ref, maskf_ref, abar_ref, out_ref):
    abar = abar_ref[:, :]
    sqrt_ab = jnp.sqrt(abar + 1e-06)
    sqrt_om = jnp.sqrt(jnp.clip(1.0 - abar, 0.0, None))
    xh_x = jnp.clip((ptx_ref[:, :] - sqrt_om * epsx_ref[:, :]) / sqrt_ab,
                    -1.0 + 0.001, 1.0 - 0.001)
    xh_y = jnp.clip((pty_ref[:, :] - sqrt_om * epsy_ref[:, :]) / sqrt_ab,
                    -1.0 + 0.001, 1.0 - 0.001)
    logit = logit_ref[:, :]
    prob = 1.0 / (1.0 + jnp.exp(-logit))
    qiota = jax.lax.broadcasted_iota(jnp.int32, (_B, _Q), 1)
    lane_iota = jax.lax.broadcasted_iota(jnp.int32, (_B, 128), 1)

    def body(t, carry):
        assigned, lx0 = carry
        base = pl.multiple_of((t // 128) * 128, 128)
        lane = t - base
        sel = lane_iota == lane

        def col(ref):
            tile = ref[:, pl.ds(base, 128)]
            return jnp.sum(jnp.where(sel, tile, 0.0), axis=1, keepdims=True)

        gx = col(p0x_ref)
        gy = col(p0y_ref)
        vm = col(maskf_ref)
        cost = jnp.abs(xh_x - gx) + jnp.abs(xh_y - gy) - prob
        c = jnp.where(assigned > 0.0, jnp.inf, cost)
        cmin = jnp.min(c, axis=1, keepdims=True)
        s = jnp.min(jnp.where(c == cmin, qiota, _Q), axis=1, keepdims=True)
        onehot = qiota == s
        valid = vm > 0.0
        assigned = jnp.maximum(assigned,
                               (onehot & valid).astype(jnp.float32))
        mx = jnp.sum(jnp.where(onehot, xh_x, 0.0), axis=1, keepdims=True)
        my = jnp.sum(jnp.where(onehot, xh_y, 0.0), axis=1, keepdims=True)
        px = (mx + 1.0) * 0.5 * 255.0
        py = (my + 1.0) * 0.5 * 255.0
        gpx = (gx + 1.0) * 0.5 * 255.0
        gpy = (gy + 1.0) * 0.5 * 255.0
        dx = px - gpx
        dy = py - gpy
        adx = jnp.abs(dx)
        ady = jnp.abs(dy)
        ex = jnp.where(adx < 1.0, 0.5 * dx * dx, adx - 0.5)
        ey = jnp.where(ady < 1.0, 0.5 * dy * dy, ady - 0.5)
        contrib = jnp.sum(jnp.where(valid, ex + ey, 0.0))
        return assigned, lx0 + contrib

    assigned0 = jnp.zeros((_B, _Q), dtype=jnp.float32)
    assigned, lx0_sum = jax.lax.fori_loop(0, _T, body, (assigned0, 0.0))

    maskf = maskf_ref[:, :]
    n = jnp.sum(maskf)
    L_x0 = jnp.where(n > 0.0, lx0_sum / jnp.maximum(2.0 * n, 1.0), 0.0)

    y = assigned
    x = logit
    ce = jnp.clip(x, 0.0, None) - x * y + jnp.log1p(jnp.exp(-jnp.abs(x)))
    p = jax.nn.sigmoid(x)
    pt = jnp.clip(jnp.where(y == 1.0, p, 1.0 - p), 1e-06, 1.0 - 1e-06)
    alpha_t = jnp.where(y == 1.0, 0.9, 1.0 - 0.9)
    omp = 1.0 - pt
    L_exist = jnp.sum(alpha_t * omp * omp * ce) * (1.0 / (_B * _Q))

    pred_cnt = jnp.sum(jax.nn.sigmoid(logit), axis=1, keepdims=True)
    gt_cnt = jnp.sum(maskf, axis=1, keepdims=True)
    L_cnt = jnp.sum(jnp.abs(pred_cnt - gt_cnt)) * (1.0 / _B)

    loss = 1.0 * L_x0 + 1.0 * L_exist + 0.1 * L_cnt
    out_ref[0] = loss
    out_ref[1] = L_exist
    out_ref[2] = L_x0
    out_ref[3] = L_cnt


def kernel(p_t, p0, mask, abar_t, eps_pred, exist_logit):
    ptx = p_t[:, :, 0]
    pty = p_t[:, :, 1]
    epsx = eps_pred[:, :, 0]
    epsy = eps_pred[:, :, 1]
    p0x = p0[:, :, 0]
    p0y = p0[:, :, 1]
    maskf = mask.astype(jnp.float32)
    abar = abar_t[:, None]

    out = pl.pallas_call(
        _loss_kernel,
        out_shape=jax.ShapeDtypeStruct((4,), jnp.float32),
        out_specs=pl.BlockSpec(memory_space=pltpu.SMEM),
    )(ptx, pty, epsx, epsy, exist_logit, p0x, p0y, maskf, abar)
    return (out[0], out[1], out[2], out[3])


# speedup vs baseline: 12.6647x; 1.0424x over previous
import jax
import jax.numpy as jnp
from jax.experimental import pallas as pl
from jax.experimental.pallas import tpu as pltpu

_B, _Q, _T = 64, 1024, 1024


def _loss_kernel(ptx_ref, pty_ref, epsx_ref, epsy_ref, logit_ref,
                 p0x_ref, p0y_ref, maskf_ref, abar_ref, out_ref):
    abar = abar_ref[:, :]
    sqrt_ab = jnp.sqrt(abar + 1e-06)
    sqrt_om = jnp.sqrt(jnp.clip(1.0 - abar, 0.0, None))
    xh_x = jnp.clip((ptx_ref[:, :] - sqrt_om * epsx_ref[:, :]) / sqrt_ab,
                    -1.0 + 0.001, 1.0 - 0.001)
    xh_y = jnp.clip((pty_ref[:, :] - sqrt_om * epsy_ref[:, :]) / sqrt_ab,
                    -1.0 + 0.001, 1.0 - 0.001)
    logit = logit_ref[:, :]
    prob = 1.0 / (1.0 + jnp.exp(-logit))
    qiota = jax.lax.broadcasted_iota(jnp.int32, (_B, _Q), 1)

    def body(t, carry):
        assigned, gxq, gyq = carry
        base = pl.multiple_of((t // 128) * 128, 128)
        lane = t - base
        shift = (128 - lane) % 128

        def col(ref):
            tile = ref[:, pl.ds(base, 128)]
            return pltpu.roll(tile, shift, axis=1)[:, 0:1]

        gx = col(p0x_ref)
        gy = col(p0y_ref)
        vm = col(maskf_ref)
        cost = jnp.abs(xh_x - gx) + jnp.abs(xh_y - gy) - prob
        c = jnp.where(assigned > 0.0, jnp.inf, cost)
        cmin = jnp.min(c, axis=1, keepdims=True)
        s = jnp.min(jnp.where(c == cmin, qiota, _Q), axis=1, keepdims=True)
        upd = (qiota == s) & (vm > 0.0)
        assigned = jnp.maximum(assigned, upd.astype(jnp.float32))
        gxq = jnp.where(upd, gx, gxq)
        gyq = jnp.where(upd, gy, gyq)
        return assigned, gxq, gyq

    zeros = jnp.zeros((_B, _Q), dtype=jnp.float32)
    assigned, gxq, gyq = jax.lax.fori_loop(
        0, _T, body, (zeros, zeros, zeros))

    px = (xh_x + 1.0) * 0.5 * 255.0
    py = (xh_y + 1.0) * 0.5 * 255.0
    gpx = (gxq + 1.0) * 0.5 * 255.0
    gpy = (gyq + 1.0) * 0.5 * 255.0
    dx = px - gpx
    dy = py - gpy
    adx = jnp.abs(dx)
    ady = jnp.abs(dy)
    ex = jnp.where(adx < 1.0, 0.5 * dx * dx, adx - 0.5)
    ey = jnp.where(ady < 1.0, 0.5 * dy * dy, ady - 0.5)
    lx0_sum = jnp.sum((ex + ey) * assigned)

    maskf = maskf_ref[:, :]
    n = jnp.sum(maskf)
    L_x0 = jnp.where(n > 0.0, lx0_sum / jnp.maximum(2.0 * n, 1.0), 0.0)

    y = assigned
    x = logit
    ce = jnp.clip(x, 0.0, None) - x * y + jnp.log1p(jnp.exp(-jnp.abs(x)))
    p = jax.nn.sigmoid(x)
    pt = jnp.clip(jnp.where(y == 1.0, p, 1.0 - p), 1e-06, 1.0 - 1e-06)
    alpha_t = jnp.where(y == 1.0, 0.9, 1.0 - 0.9)
    omp = 1.0 - pt
    L_exist = jnp.sum(alpha_t * omp * omp * ce) * (1.0 / (_B * _Q))

    pred_cnt = jnp.sum(jax.nn.sigmoid(logit), axis=1, keepdims=True)
    gt_cnt = jnp.sum(maskf, axis=1, keepdims=True)
    L_cnt = jnp.sum(jnp.abs(pred_cnt - gt_cnt)) * (1.0 / _B)

    loss = 1.0 * L_x0 + 1.0 * L_exist + 0.1 * L_cnt
    out_ref[0] = loss
    out_ref[1] = L_exist
    out_ref[2] = L_x0
    out_ref[3] = L_cnt


def kernel(p_t, p0, mask, abar_t, eps_pred, exist_logit):
    ptx = p_t[:, :, 0]
    pty = p_t[:, :, 1]
    epsx = eps_pred[:, :, 0]
    epsy = eps_pred[:, :, 1]
    p0x = p0[:, :, 0]
    p0y = p0[:, :, 1]
    maskf = mask.astype(jnp.float32)
    abar = abar_t[:, None]

    out = pl.pallas_call(
        _loss_kernel,
        out_shape=jax.ShapeDtypeStruct((4,), jnp.float32),
        out_specs=pl.BlockSpec(memory_space=pltpu.SMEM),
    )(ptx, pty, epsx, epsy, exist_logit, p0x, p0y, maskf, abar)
    return (out[0], out[1], out[2], out[3])


# speedup vs baseline: 15.2635x; 1.2052x over previous
import jax
import jax.numpy as jnp
from jax.experimental import pallas as pl
from jax.experimental.pallas import tpu as pltpu

_B, _Q, _T = 64, 1024, 1024


def _loss_kernel(ptx_ref, pty_ref, epsx_ref, epsy_ref, logit_ref,
                 p0x_ref, p0y_ref, maskf_ref, abar_ref, out_ref,
                 xhx_ref, xhy_ref, pm_ref, gxq_ref, gyq_ref):
    abar = abar_ref[:, :]
    sqrt_ab = jnp.sqrt(abar + 1e-06)
    sqrt_om = jnp.sqrt(jnp.clip(1.0 - abar, 0.0, None))
    xhx_ref[:, :] = jnp.clip(
        (ptx_ref[:, :] - sqrt_om * epsx_ref[:, :]) / sqrt_ab,
        -1.0 + 0.001, 1.0 - 0.001)
    xhy_ref[:, :] = jnp.clip(
        (pty_ref[:, :] - sqrt_om * epsy_ref[:, :]) / sqrt_ab,
        -1.0 + 0.001, 1.0 - 0.001)
    logit = logit_ref[:, :]
    pm_ref[:, :] = 1.0 / (1.0 + jnp.exp(-logit))
    zeros = jnp.zeros((_B, _Q), dtype=jnp.float32)
    gxq_ref[:, :] = zeros
    gyq_ref[:, :] = zeros
    qiota = jax.lax.broadcasted_iota(jnp.int32, (_B, _Q), 1)

    def body(t, carry):
        base = pl.multiple_of((t // 128) * 128, 128)
        lane = t - base
        shift = (128 - lane) % 128

        def col(ref):
            tile = ref[:, pl.ds(base, 128)]
            return pltpu.roll(tile, shift, axis=1)[:, 0:1]

        gx = col(p0x_ref)
        gy = col(p0y_ref)
        vm = col(maskf_ref)
        pm = pm_ref[:, :]
        cost = (jnp.abs(xhx_ref[:, :] - gx)
                + jnp.abs(xhy_ref[:, :] - gy) - pm)
        cmin = jnp.min(cost, axis=1, keepdims=True)
        s = jnp.min(jnp.where(cost == cmin, qiota, _Q),
                    axis=1, keepdims=True)
        upd = (qiota == s) & (vm > 0.0)
        pm_ref[:, :] = jnp.where(upd, -jnp.inf, pm)
        gxq_ref[:, :] = jnp.where(upd, gx, gxq_ref[:, :])
        gyq_ref[:, :] = jnp.where(upd, gy, gyq_ref[:, :])
        return carry

    jax.lax.fori_loop(0, _T, body, 0, unroll=False)

    assigned = (pm_ref[:, :] == -jnp.inf).astype(jnp.float32)

    px = (xhx_ref[:, :] + 1.0) * 0.5 * 255.0
    py = (xhy_ref[:, :] + 1.0) * 0.5 * 255.0
    gpx = (gxq_ref[:, :] + 1.0) * 0.5 * 255.0
    gpy = (gyq_ref[:, :] + 1.0) * 0.5 * 255.0
    dx = px - gpx
    dy = py - gpy
    adx = jnp.abs(dx)
    ady = jnp.abs(dy)
    ex = jnp.where(adx < 1.0, 0.5 * dx * dx, adx - 0.5)
    ey = jnp.where(ady < 1.0, 0.5 * dy * dy, ady - 0.5)
    lx0_sum = jnp.sum((ex + ey) * assigned)

    maskf = maskf_ref[:, :]
    n = jnp.sum(maskf)
    L_x0 = jnp.where(n > 0.0, lx0_sum / jnp.maximum(2.0 * n, 1.0), 0.0)

    y = assigned
    x = logit
    ce = jnp.clip(x, 0.0, None) - x * y + jnp.log1p(jnp.exp(-jnp.abs(x)))
    p = jax.nn.sigmoid(x)
    pt = jnp.clip(jnp.where(y == 1.0, p, 1.0 - p), 1e-06, 1.0 - 1e-06)
    alpha_t = jnp.where(y == 1.0, 0.9, 1.0 - 0.9)
    omp = 1.0 - pt
    L_exist = jnp.sum(alpha_t * omp * omp * ce) * (1.0 / (_B * _Q))

    pred_cnt = jnp.sum(jax.nn.sigmoid(logit), axis=1, keepdims=True)
    gt_cnt = jnp.sum(maskf, axis=1, keepdims=True)
    L_cnt = jnp.sum(jnp.abs(pred_cnt - gt_cnt)) * (1.0 / _B)

    loss = 1.0 * L_x0 + 1.0 * L_exist + 0.1 * L_cnt
    out_ref[0] = loss
    out_ref[1] = L_exist
    out_ref[2] = L_x0
    out_ref[3] = L_cnt


def kernel(p_t, p0, mask, abar_t, eps_pred, exist_logit):
    ptx = p_t[:, :, 0]
    pty = p_t[:, :, 1]
    epsx = eps_pred[:, :, 0]
    epsy = eps_pred[:, :, 1]
    p0x = p0[:, :, 0]
    p0y = p0[:, :, 1]
    maskf = mask.astype(jnp.float32)
    abar = abar_t[:, None]

    out = pl.pallas_call(
        _loss_kernel,
        out_shape=jax.ShapeDtypeStruct((4,), jnp.float32),
        out_specs=pl.BlockSpec(memory_space=pltpu.SMEM),
        scratch_shapes=[pltpu.VMEM((_B, _Q), jnp.float32)] * 5,
    )(ptx, pty, epsx, epsy, exist_logit, p0x, p0y, maskf, abar)
    return (out[0], out[1], out[2], out[3])
